# Initial kernel scaffold; baseline (speedup 1.0000x reference)
import jax
import jax.numpy as jnp
from jax.experimental import pallas as pl


def kernel(x, rms_w, Wr, Ws1, Ws2, We1, We2):
    raise NotImplementedError("write your pallas kernel here")


# speedup vs baseline: 2.0255x; 2.0255x over previous
import functools

import jax
import jax.numpy as jnp
import numpy as np
from jax.experimental import pallas as pl
from jax.experimental.pallas import tpu as pltpu

B_ = 1
S_ = 2048
D_ = 1024
F_ = 512
E_ = 8
K_ = 2
T_ = B_ * S_
C_ = int(np.ceil(T_ * K_ / E_ * 2.0))
EPS_ = 1e-6
TB_ = 256


def _k1_router(x_ref, w_ref, wrt_ref, xn_ref, c_ref, cnt_ref):
    i = pl.program_id(0)

    @pl.when(i == 0)
    def _():
        cnt_ref[...] = jnp.zeros_like(cnt_ref)

    xt = x_ref[...]
    ms = jnp.mean(xt * xt, axis=-1, keepdims=True)
    xn = xt * jax.lax.rsqrt(ms + EPS_) * w_ref[...]
    xn_ref[...] = xn

    logits = jnp.dot(xn.astype(jnp.bfloat16), wrt_ref[...],
                     preferred_element_type=jnp.float32)
    logits = logits.astype(jnp.bfloat16).astype(jnp.float32)
    probs = jax.nn.softmax(logits, axis=-1)

    idx = jax.lax.broadcasted_iota(jnp.int32, (TB_, E_), 1)
    m1 = jnp.max(probs, axis=-1, keepdims=True)
    i1 = jnp.min(jnp.where(probs == m1, idx, E_), axis=-1, keepdims=True)
    oh1 = (idx == i1)
    pm = jnp.where(oh1, -jnp.inf, probs)
    m2 = jnp.max(pm, axis=-1, keepdims=True)
    i2 = jnp.min(jnp.where(pm == m2, idx, E_), axis=-1, keepdims=True)
    oh2 = (idx == i2)
    denom = m1 + m2
    g1 = m1 / denom
    g2 = m2 / denom

    oh1f = oh1.astype(jnp.float32)
    oh2f = oh2.astype(jnp.float32)
    oht = oh1f + oh2f

    r = jax.lax.broadcasted_iota(jnp.int32, (TB_, TB_), 0)
    cmask = (r > jax.lax.broadcasted_iota(jnp.int32, (TB_, TB_), 1))
    excl = jnp.dot(cmask.astype(jnp.bfloat16), oht.astype(jnp.bfloat16),
                   preferred_element_type=jnp.float32)

    base = cnt_ref[...]
    pref = base + excl
    pos1 = jnp.sum(oh1f * pref, axis=-1, keepdims=True)
    pos2 = jnp.sum(oh2f * pref, axis=-1, keepdims=True)
    keep1 = (pos1 < C_).astype(jnp.float32)
    keep2 = (pos2 < C_).astype(jnp.float32)
    c_ref[...] = oh1f * (g1 * keep1) + oh2f * (g2 * keep2)
    cnt_ref[...] = base + jnp.sum(oht, axis=0, keepdims=True)


def _k2_moe(xn_ref, c_ref, ws1t_ref, ws2t_ref, we1t_ref, we2t_ref, o_ref):
    xt = xn_ref[...]
    h = jax.nn.silu(jnp.dot(xt, ws1t_ref[...],
                            preferred_element_type=jnp.float32))
    acc = jnp.dot(h, ws2t_ref[...], preferred_element_type=jnp.float32)
    c = c_ref[...]
    for e in range(E_):
        he = jax.nn.silu(jnp.dot(xt, we1t_ref[e],
                                 preferred_element_type=jnp.float32))
        acc = acc + c[:, e:e + 1] * jnp.dot(he, we2t_ref[e],
                                            preferred_element_type=jnp.float32)
    o_ref[...] = acc


def kernel(x, rms_w, Wr, Ws1, Ws2, We1, We2):
    Bp, Sp, d = x.shape
    flat = x.reshape(T_, D_)
    wrt = Wr.T.astype(jnp.bfloat16)
    ws1t = Ws1.T
    ws2t = Ws2.T
    we1t = jnp.transpose(We1, (0, 2, 1))
    we2t = jnp.transpose(We2, (0, 2, 1))

    nt = T_ // TB_
    xn, c = pl.pallas_call(
        _k1_router,
        grid=(nt,),
        in_specs=[
            pl.BlockSpec((TB_, D_), lambda i: (i, 0)),
            pl.BlockSpec((1, D_), lambda i: (0, 0)),
            pl.BlockSpec((D_, E_), lambda i: (0, 0)),
        ],
        out_specs=[
            pl.BlockSpec((TB_, D_), lambda i: (i, 0)),
            pl.BlockSpec((TB_, E_), lambda i: (i, 0)),
        ],
        out_shape=[
            jax.ShapeDtypeStruct((T_, D_), jnp.float32),
            jax.ShapeDtypeStruct((T_, E_), jnp.float32),
        ],
        scratch_shapes=[pltpu.VMEM((1, E_), jnp.float32)],
    )(flat, rms_w.reshape(1, D_), wrt)

    out = pl.pallas_call(
        _k2_moe,
        grid=(nt,),
        in_specs=[
            pl.BlockSpec((TB_, D_), lambda i: (i, 0)),
            pl.BlockSpec((TB_, E_), lambda i: (i, 0)),
            pl.BlockSpec((D_, F_), lambda i: (0, 0)),
            pl.BlockSpec((F_, D_), lambda i: (0, 0)),
            pl.BlockSpec((E_, D_, F_), lambda i: (0, 0, 0)),
            pl.BlockSpec((E_, F_, D_), lambda i: (0, 0, 0)),
        ],
        out_specs=pl.BlockSpec((TB_, D_), lambda i: (i, 0)),
        out_shape=jax.ShapeDtypeStruct((T_, D_), jnp.float32),
    )(xn, c, ws1t, ws2t, we1t, we2t)

    return out.reshape(Bp, Sp, d)


# speedup vs baseline: 2.1747x; 1.0737x over previous
import functools

import jax
import jax.numpy as jnp
import numpy as np
from jax.experimental import pallas as pl
from jax.experimental.pallas import tpu as pltpu

B_ = 1
S_ = 2048
D_ = 1024
F_ = 512
E_ = 8
K_ = 2
T_ = B_ * S_
C_ = int(np.ceil(T_ * K_ / E_ * 2.0))
EPS_ = 1e-6
TB_ = 256


def _k1_router(x_ref, w_ref, wrt_ref, xn_ref, c_ref, cnt_ref):
    i = pl.program_id(0)

    @pl.when(i == 0)
    def _():
        cnt_ref[...] = jnp.zeros_like(cnt_ref)

    xt = x_ref[...]
    ms = jnp.mean(xt * xt, axis=-1, keepdims=True)
    xn = xt * jax.lax.rsqrt(ms + EPS_) * w_ref[...]
    xnb = xn.astype(jnp.bfloat16)
    xn_ref[...] = xnb

    logits = jnp.dot(xnb, wrt_ref[...],
                     preferred_element_type=jnp.float32)
    logits = logits.astype(jnp.bfloat16).astype(jnp.float32)
    probs = jax.nn.softmax(logits, axis=-1)

    idx = jax.lax.broadcasted_iota(jnp.int32, (TB_, E_), 1)
    m1 = jnp.max(probs, axis=-1, keepdims=True)
    i1 = jnp.min(jnp.where(probs == m1, idx, E_), axis=-1, keepdims=True)
    oh1 = (idx == i1)
    pm = jnp.where(oh1, -jnp.inf, probs)
    m2 = jnp.max(pm, axis=-1, keepdims=True)
    i2 = jnp.min(jnp.where(pm == m2, idx, E_), axis=-1, keepdims=True)
    oh2 = (idx == i2)
    denom = m1 + m2
    g1 = m1 / denom
    g2 = m2 / denom

    oh1f = oh1.astype(jnp.float32)
    oh2f = oh2.astype(jnp.float32)
    oht = oh1f + oh2f

    r = jax.lax.broadcasted_iota(jnp.int32, (TB_, TB_), 0)
    cmask = (r > jax.lax.broadcasted_iota(jnp.int32, (TB_, TB_), 1))
    excl = jnp.dot(cmask.astype(jnp.bfloat16), oht.astype(jnp.bfloat16),
                   preferred_element_type=jnp.float32)

    base = cnt_ref[...]
    pref = base + excl
    pos1 = jnp.sum(oh1f * pref, axis=-1, keepdims=True)
    pos2 = jnp.sum(oh2f * pref, axis=-1, keepdims=True)
    keep1 = (pos1 < C_).astype(jnp.float32)
    keep2 = (pos2 < C_).astype(jnp.float32)
    c_ref[...] = oh1f * (g1 * keep1) + oh2f * (g2 * keep2)
    cnt_ref[...] = base + jnp.sum(oht, axis=0, keepdims=True)


def _k2_moe(xn_ref, c_ref, ws1t_ref, ws2t_ref, we1t_ref, we2t_ref, o_ref):
    xt = xn_ref[...]
    h = jax.nn.silu(jnp.dot(xt, ws1t_ref[...],
                            preferred_element_type=jnp.float32))
    acc = jnp.dot(h.astype(jnp.bfloat16), ws2t_ref[...],
                  preferred_element_type=jnp.float32)
    c = c_ref[...]
    for e in range(E_):
        he = jax.nn.silu(jnp.dot(xt, we1t_ref[e],
                                 preferred_element_type=jnp.float32))
        acc = acc + c[:, e:e + 1] * jnp.dot(he.astype(jnp.bfloat16),
                                            we2t_ref[e],
                                            preferred_element_type=jnp.float32)
    o_ref[...] = acc


def kernel(x, rms_w, Wr, Ws1, Ws2, We1, We2):
    Bp, Sp, d = x.shape
    flat = x.reshape(T_, D_)
    wrt = Wr.T.astype(jnp.bfloat16)
    ws1t = Ws1.T.astype(jnp.bfloat16)
    ws2t = Ws2.T.astype(jnp.bfloat16)
    we1t = jnp.transpose(We1, (0, 2, 1)).astype(jnp.bfloat16)
    we2t = jnp.transpose(We2, (0, 2, 1)).astype(jnp.bfloat16)

    nt = T_ // TB_
    xn, c = pl.pallas_call(
        _k1_router,
        grid=(nt,),
        in_specs=[
            pl.BlockSpec((TB_, D_), lambda i: (i, 0)),
            pl.BlockSpec((1, D_), lambda i: (0, 0)),
            pl.BlockSpec((D_, E_), lambda i: (0, 0)),
        ],
        out_specs=[
            pl.BlockSpec((TB_, D_), lambda i: (i, 0)),
            pl.BlockSpec((TB_, E_), lambda i: (i, 0)),
        ],
        out_shape=[
            jax.ShapeDtypeStruct((T_, D_), jnp.bfloat16),
            jax.ShapeDtypeStruct((T_, E_), jnp.float32),
        ],
        scratch_shapes=[pltpu.VMEM((1, E_), jnp.float32)],
    )(flat, rms_w.reshape(1, D_), wrt)

    out = pl.pallas_call(
        _k2_moe,
        grid=(nt,),
        in_specs=[
            pl.BlockSpec((TB_, D_), lambda i: (i, 0)),
            pl.BlockSpec((TB_, E_), lambda i: (i, 0)),
            pl.BlockSpec((D_, F_), lambda i: (0, 0)),
            pl.BlockSpec((F_, D_), lambda i: (0, 0)),
            pl.BlockSpec((E_, D_, F_), lambda i: (0, 0, 0)),
            pl.BlockSpec((E_, F_, D_), lambda i: (0, 0, 0)),
        ],
        out_specs=pl.BlockSpec((TB_, D_), lambda i: (i, 0)),
        out_shape=jax.ShapeDtypeStruct((T_, D_), jnp.float32),
    )(xn, c, ws1t, ws2t, we1t, we2t)

    return out.reshape(Bp, Sp, d)


# speedup vs baseline: 3.2110x; 1.4765x over previous
import jax
import jax.numpy as jnp
import numpy as np
from jax.experimental import pallas as pl
from jax.experimental.pallas import tpu as pltpu

B_ = 1
S_ = 2048
D_ = 1024
F_ = 512
E_ = 8
K_ = 2
T_ = B_ * S_
C_ = int(np.ceil(T_ * K_ / E_ * 2.0))
EPS_ = 1e-6
TB_ = 256


def _dot_t(a, b):
    return jax.lax.dot_general(a, b, (((1,), (1,)), ((), ())),
                               preferred_element_type=jnp.float32)


def _moe_fused(x_ref, w_ref, wr_ref, ws1_ref, ws2_ref, we1_ref, we2_ref,
               o_ref, cnt_ref):
    i = pl.program_id(0)

    @pl.when(i == 0)
    def _():
        cnt_ref[...] = jnp.zeros_like(cnt_ref)

    xt = x_ref[...]
    ms = jnp.mean(xt * xt, axis=-1, keepdims=True)
    xn = xt * jax.lax.rsqrt(ms + EPS_) * w_ref[...]

    logits = _dot_t(xn.astype(jnp.bfloat16), wr_ref[...].astype(jnp.bfloat16))
    logits = logits.astype(jnp.bfloat16).astype(jnp.float32)
    probs = jax.nn.softmax(logits, axis=-1)

    idx = jax.lax.broadcasted_iota(jnp.int32, (TB_, E_), 1)
    m1 = jnp.max(probs, axis=-1, keepdims=True)
    i1 = jnp.min(jnp.where(probs == m1, idx, E_), axis=-1, keepdims=True)
    oh1 = (idx == i1)
    pm = jnp.where(oh1, -jnp.inf, probs)
    m2 = jnp.max(pm, axis=-1, keepdims=True)
    i2 = jnp.min(jnp.where(pm == m2, idx, E_), axis=-1, keepdims=True)
    oh2 = (idx == i2)
    denom = m1 + m2
    g1 = m1 / denom
    g2 = m2 / denom

    oh1f = oh1.astype(jnp.float32)
    oh2f = oh2.astype(jnp.float32)
    oht = oh1f + oh2f

    r = jax.lax.broadcasted_iota(jnp.int32, (TB_, TB_), 0)
    cmask = (r > jax.lax.broadcasted_iota(jnp.int32, (TB_, TB_), 1))
    excl = jnp.dot(cmask.astype(jnp.bfloat16), oht.astype(jnp.bfloat16),
                   preferred_element_type=jnp.float32)

    base = cnt_ref[...]
    pref = base + excl
    pos1 = jnp.sum(oh1f * pref, axis=-1, keepdims=True)
    pos2 = jnp.sum(oh2f * pref, axis=-1, keepdims=True)
    keep1 = (pos1 < C_).astype(jnp.float32)
    keep2 = (pos2 < C_).astype(jnp.float32)
    c = oh1f * (g1 * keep1) + oh2f * (g2 * keep2)
    cnt_ref[...] = base + jnp.sum(oht, axis=0, keepdims=True)

    h = jax.nn.silu(_dot_t(xn, ws1_ref[...]))
    acc = _dot_t(h, ws2_ref[...])
    for e in range(E_):
        he = jax.nn.silu(_dot_t(xn, we1_ref[e]))
        acc = acc + c[:, e:e + 1] * _dot_t(he, we2_ref[e])
    o_ref[...] = acc


def kernel(x, rms_w, Wr, Ws1, Ws2, We1, We2):
    Bp, Sp, d = x.shape
    flat = x.reshape(T_, D_)

    nt = T_ // TB_
    out = pl.pallas_call(
        _moe_fused,
        grid=(nt,),
        in_specs=[
            pl.BlockSpec((TB_, D_), lambda i: (i, 0)),
            pl.BlockSpec((1, D_), lambda i: (0, 0)),
            pl.BlockSpec((E_, D_), lambda i: (0, 0)),
            pl.BlockSpec((F_, D_), lambda i: (0, 0)),
            pl.BlockSpec((D_, F_), lambda i: (0, 0)),
            pl.BlockSpec((E_, F_, D_), lambda i: (0, 0, 0)),
            pl.BlockSpec((E_, D_, F_), lambda i: (0, 0, 0)),
        ],
        out_specs=pl.BlockSpec((TB_, D_), lambda i: (i, 0)),
        out_shape=jax.ShapeDtypeStruct((T_, D_), jnp.float32),
        scratch_shapes=[pltpu.VMEM((1, E_), jnp.float32)],
    )(flat, rms_w.reshape(1, D_), Wr, Ws1, Ws2, We1, We2)

    return out.reshape(Bp, Sp, d)
